# baseline (speedup 1.0000x reference)
import jax
import jax.numpy as jnp
from jax import lax
from jax.experimental import pallas as pl
from jax.experimental.pallas import tpu as pltpu
from jax.experimental.pallas import tpu_sc as plsc

_NC = 2
_NS = 16
_NW = _NC * _NS
_L = 16
_C = 256
_NB = 4


def _transpose_table_body(iv_ref, out_ref):
    out_ref[...] = iv_ref[...].T


def _transpose_table(iv):
    dim, vocab = iv.shape
    blk = 4096
    return pl.pallas_call(
        _transpose_table_body,
        grid=(pl.cdiv(vocab, blk),),
        in_specs=[pl.BlockSpec((dim, blk), lambda i: (0, i))],
        out_specs=pl.BlockSpec((blk, dim), lambda i: (i, 0)),
        out_shape=jax.ShapeDtypeStruct((vocab, dim), jnp.float32),
    )(iv)


def _relayout_out_body(rows_ref, out_ref):
    out_ref[...] = rows_ref[...].swapaxes(1, 2)


def _relayout_out(rows, seq, batch):
    n, dim = rows.shape
    rows3 = rows.reshape(seq, batch, dim)
    return pl.pallas_call(
        _relayout_out_body,
        grid=(seq,),
        in_specs=[pl.BlockSpec((1, batch, dim), lambda s: (s, 0, 0))],
        out_specs=pl.BlockSpec((1, dim, batch), lambda s: (s, 0, 0)),
        out_shape=jax.ShapeDtypeStruct((seq, dim, batch), jnp.float32),
    )(rows3)


def _gather_body(idx_hbm, table_hbm, out_hbm, idx_v, rows_v, sem_i, sem_g, sem_o):
    n_total = idx_hbm.shape[0]
    n_per_w = n_total // _NW
    n_chunks = n_per_w // _C
    wid = lax.axis_index("s") * _NC + lax.axis_index("c")
    base = wid * n_per_w

    for b in range(_NB):
        pltpu.async_copy(
            idx_hbm.at[pl.ds(base + b * _C, _C)], idx_v.at[b], sem_i.at[b])

    @pl.loop(0, n_chunks, step=_NB)
    def _(j0):
        for b in range(_NB):
            pltpu.make_async_copy(
                idx_hbm.at[pl.ds(base, _C)], idx_v.at[b], sem_i.at[b]).wait()

            @pl.loop(0, _C, step=_L)
            def _(r):
                vals = idx_v[b, pl.ds(r, _L)]
                pltpu.async_copy(
                    table_hbm.at[vals], rows_v.at[b, pl.ds(r, _L)],
                    sem_g.at[b])
        stores = []
        for b in range(_NB):
            pltpu.make_async_copy(
                table_hbm.at[pl.ds(0, _C)], rows_v.at[b], sem_g.at[b]).wait()
            stores.append(pltpu.async_copy(
                rows_v.at[b],
                out_hbm.at[pl.ds(base + (j0 + b) * _C, _C)], sem_o.at[b]))
            off_next = base + lax.rem(j0 + _NB + b, n_chunks) * _C
            pltpu.async_copy(
                idx_hbm.at[pl.ds(off_next, _C)], idx_v.at[b], sem_i.at[b])
        for b in range(_NB):
            stores[b].wait()

    for b in range(_NB):
        pltpu.make_async_copy(
            idx_hbm.at[pl.ds(base, _C)], idx_v.at[b], sem_i.at[b]).wait()


def _sc_gather(idx, table_rm):
    n_total, = idx.shape
    _, dim = table_rm.shape
    mesh = plsc.VectorSubcoreMesh(core_axis_name="c", subcore_axis_name="s")
    return pl.kernel(
        _gather_body,
        out_type=jax.ShapeDtypeStruct((n_total, dim), jnp.float32),
        mesh=mesh,
        scratch_types=[
            pltpu.VMEM((_NB, _C), jnp.int32),
            pltpu.VMEM((_NB, _C, dim), jnp.float32),
            pltpu.SemaphoreType.DMA((_NB,)),
            pltpu.SemaphoreType.DMA((_NB,)),
            pltpu.SemaphoreType.DMA((_NB,)),
        ],
        compiler_params=pltpu.CompilerParams(use_tc_tiling_on_sc=False),
    )(idx, table_rm)


def kernel(x, src_weight):
    batch, seq = x.shape
    _, dim = src_weight.shape
    idx = x.T.reshape(batch * seq).astype(jnp.int32)
    table_rm = _transpose_table(src_weight.T)
    rows = _sc_gather(idx, table_rm)
    out_p = _relayout_out(rows, seq, batch)
    return out_p.transpose(2, 0, 1)


# speedup vs baseline: 1.1853x; 1.1853x over previous
import jax
import jax.numpy as jnp
from jax import lax
from jax.experimental import pallas as pl
from jax.experimental.pallas import tpu as pltpu
from jax.experimental.pallas import tpu_sc as plsc

_NC = 2
_NS = 16
_NW = _NC * _NS
_L = 16
_C = 256
_NB = 4


def _gather_body(idx_hbm, table_hbm, out_hbm, idx_v, rows_v, sem_i, sem_g, sem_o):
    n_total = idx_hbm.shape[0]
    n_per_w = n_total // _NW
    n_chunks = n_per_w // _C
    wid = lax.axis_index("s") * _NC + lax.axis_index("c")
    base = wid * n_per_w

    def drain_gathers(b):
        pltpu.make_async_copy(
            table_hbm.at[pl.ds(0, _C)], rows_v.at[b], sem_g.at[b]).wait()

    for b in range(_NB):
        pltpu.async_copy(
            idx_hbm.at[pl.ds(base + b * _C, _C)], idx_v.at[b], sem_i.at[b])

    @pl.loop(0, n_chunks, step=_NB)
    def _(j0):
        for b in range(_NB):
            pltpu.make_async_copy(
                idx_hbm.at[pl.ds(base, _C)], idx_v.at[b], sem_i.at[b]).wait()

            @pl.loop(0, _C, step=_L)
            def _(r):
                vals = idx_v[b, pl.ds(r, _L)]
                pltpu.async_copy(
                    table_hbm.at[vals], rows_v.at[b, pl.ds(r, _L)],
                    sem_g.at[b])
        stores = []
        for b in range(_NB):
            drain_gathers(b)
            stores.append(pltpu.async_copy(
                rows_v.at[b],
                out_hbm.at[pl.ds(base + (j0 + b) * _C, _C)], sem_o.at[b]))
            off_next = base + lax.rem(j0 + _NB + b, n_chunks) * _C
            pltpu.async_copy(
                idx_hbm.at[pl.ds(off_next, _C)], idx_v.at[b], sem_i.at[b])
        for b in range(_NB):
            stores[b].wait()

    for b in range(_NB):
        pltpu.make_async_copy(
            idx_hbm.at[pl.ds(base, _C)], idx_v.at[b], sem_i.at[b]).wait()


def kernel(x, src_weight):
    batch, seq = x.shape
    _, dim = src_weight.shape
    n_total = batch * seq
    idx = x.reshape(n_total).astype(jnp.int32)

    mesh = plsc.VectorSubcoreMesh(core_axis_name="c", subcore_axis_name="s")
    out = pl.kernel(
        _gather_body,
        out_type=jax.ShapeDtypeStruct((n_total, dim), jnp.float32),
        mesh=mesh,
        scratch_types=[
            pltpu.VMEM((_NB, _C), jnp.int32),
            pltpu.VMEM((_NB, _C, dim), jnp.float32),
            pltpu.SemaphoreType.DMA((_NB,)),
            pltpu.SemaphoreType.DMA((_NB,)),
            pltpu.SemaphoreType.DMA((_NB,)),
        ],
        compiler_params=pltpu.CompilerParams(use_tc_tiling_on_sc=False),
    )(idx, src_weight)
    return out.reshape(batch, seq, dim)
